# baseline (speedup 1.0000x reference)
import jax
import jax.numpy as jnp
from jax.experimental import pallas as pl
from jax.experimental.pallas import tpu as pltpu

_N = 16384
_C = 100
_T1 = 1001
_TB = 91
_RB = 2048
_TS = 256


def _seg_body(off_ref, x0s_ref, q_ref, probs_ref):
    step = pl.program_id(0)

    def seg(j, carry):
        t = step * _TB + j
        r0 = off_ref[t]
        r1 = off_ref[t + 1]
        q = q_ref[j]
        start = (r0 // _TS) * _TS
        ntiles = (r1 - start + _TS - 1) // _TS

        def tile(k, c2):
            base = start + k * _TS
            x8 = x0s_ref[pl.ds(base, _TS), :]
            p8 = jnp.dot(x8, q, preferred_element_type=jnp.float32)
            rows = base + jax.lax.broadcasted_iota(jnp.int32, (_TS, 1), 0)
            mask = (rows >= r0) & (rows < r1)
            old = probs_ref[pl.ds(base, _TS), :]
            probs_ref[pl.ds(base, _TS), :] = jnp.where(mask, p8, old)
            return c2

        jax.lax.fori_loop(0, ntiles, tile, 0)
        return carry

    jax.lax.fori_loop(0, _TB, seg, 0)


def _sample_body(probs_ref, g_ref, oh_ref):
    p = probs_ref[...]
    pn = p / jnp.sum(p, axis=1, keepdims=True)
    y = jnp.log(jnp.maximum(pn, 1e-30)) + g_ref[...]
    s = jnp.argmax(y, axis=1)
    oh_ref[...] = (jax.lax.broadcasted_iota(jnp.int32, (_RB, _C), 1)
                   == s[:, None]).astype(jnp.float32)


def kernel(x0_batch, time_batch, accumulated_q_matrices):
    t32 = time_batch.astype(jnp.int32)
    gnoise = jax.random.gumbel(jax.random.key(1), (_N, _C), jnp.float32)

    perm = jnp.argsort(t32)
    x0s = jnp.take(x0_batch, perm, axis=0)
    hist = jnp.zeros((_T1,), jnp.int32).at[t32].add(1)
    off = jnp.concatenate([jnp.zeros((1,), jnp.int32),
                           jnp.cumsum(hist, dtype=jnp.int32)])
    inv = jnp.zeros((_N,), jnp.int32).at[perm].set(
        jnp.arange(_N, dtype=jnp.int32))

    probs_s = pl.pallas_call(
        _seg_body,
        grid=(_T1 // _TB,),
        in_specs=[
            pl.BlockSpec((_T1 + 1,), lambda s: (0,), memory_space=pltpu.SMEM),
            pl.BlockSpec((_N, _C), lambda s: (0, 0)),
            pl.BlockSpec((_TB, _C, _C), lambda s: (s, 0, 0)),
        ],
        out_specs=pl.BlockSpec((_N, _C), lambda s: (0, 0)),
        out_shape=jax.ShapeDtypeStruct((_N, _C), jnp.float32),
    )(off, x0s, accumulated_q_matrices)

    probs = jnp.take(probs_s, inv, axis=0)

    onehot = pl.pallas_call(
        _sample_body,
        grid=(_N // _RB,),
        in_specs=[
            pl.BlockSpec((_RB, _C), lambda i: (i, 0)),
            pl.BlockSpec((_RB, _C), lambda i: (i, 0)),
        ],
        out_specs=pl.BlockSpec((_RB, _C), lambda i: (i, 0)),
        out_shape=jax.ShapeDtypeStruct((_N, _C), jnp.float32),
    )(probs, gnoise)
    return probs, onehot


# speedup vs baseline: 1.2292x; 1.2292x over previous
import jax
import jax.numpy as jnp
from jax.experimental import pallas as pl
from jax.experimental.pallas import tpu as pltpu

_N = 16384
_C = 100
_T1 = 1001
_TB = 91
_RB = 2048
_TS = 256


def _seg_body(off_ref, x0s_ref, q_ref, probs_ref):
    step = pl.program_id(0)

    def segpair(jj, carry):
        j0 = 2 * jj
        j1 = jnp.minimum(j0 + 1, _TB - 1)
        valid1 = (j0 + 1) < _TB
        t0 = step * _TB + j0
        t1 = step * _TB + j1
        r00 = off_ref[t0]
        r01 = off_ref[t0 + 1]
        r10 = off_ref[t1]
        r11 = jnp.where(valid1, off_ref[t1 + 1], r10)
        q0 = q_ref[j0]
        q1 = q_ref[j1]
        start = (r00 // _TS) * _TS
        end = jnp.maximum(r01, r11)
        ntiles = (end - start + _TS - 1) // _TS

        def tile(k, c2):
            base = start + k * _TS
            xt = x0s_ref[pl.ds(base, _TS), :]
            p0 = jnp.dot(xt, q0, preferred_element_type=jnp.float32)
            p1 = jnp.dot(xt, q1, preferred_element_type=jnp.float32)
            rows = base + jax.lax.broadcasted_iota(jnp.int32, (_TS, 1), 0)
            m0 = (rows >= r00) & (rows < r01)
            m1 = (rows >= r10) & (rows < r11)
            old = probs_ref[pl.ds(base, _TS), :]
            probs_ref[pl.ds(base, _TS), :] = jnp.where(
                m0, p0, jnp.where(m1, p1, old))
            return c2

        jax.lax.fori_loop(0, ntiles, tile, 0)
        return carry

    jax.lax.fori_loop(0, (_TB + 1) // 2, segpair, 0)


def _sample_body(probs_ref, g_ref, oh_ref):
    p = probs_ref[...]
    pn = p / jnp.sum(p, axis=1, keepdims=True)
    y = jnp.log(jnp.maximum(pn, 1e-30)) + g_ref[...]
    s = jnp.argmax(y, axis=1)
    oh_ref[...] = (jax.lax.broadcasted_iota(jnp.int32, (_RB, _C), 1)
                   == s[:, None]).astype(jnp.float32)


def kernel(x0_batch, time_batch, accumulated_q_matrices):
    t32 = time_batch.astype(jnp.int32)
    gnoise = jax.random.gumbel(jax.random.key(1), (_N, _C), jnp.float32)

    perm = jnp.argsort(t32)
    x0s = jnp.take(x0_batch, perm, axis=0)
    hist = jnp.zeros((_T1,), jnp.int32).at[t32].add(1)
    off = jnp.concatenate([jnp.zeros((1,), jnp.int32),
                           jnp.cumsum(hist, dtype=jnp.int32)])
    inv = jnp.zeros((_N,), jnp.int32).at[perm].set(
        jnp.arange(_N, dtype=jnp.int32))

    probs_s = pl.pallas_call(
        _seg_body,
        grid=(_T1 // _TB,),
        in_specs=[
            pl.BlockSpec((_T1 + 1,), lambda s: (0,), memory_space=pltpu.SMEM),
            pl.BlockSpec((_N, _C), lambda s: (0, 0)),
            pl.BlockSpec((_TB, _C, _C), lambda s: (s, 0, 0)),
        ],
        out_specs=pl.BlockSpec((_N, _C), lambda s: (0, 0)),
        out_shape=jax.ShapeDtypeStruct((_N, _C), jnp.float32),
    )(off, x0s, accumulated_q_matrices)

    probs = jnp.take(probs_s, inv, axis=0)

    onehot = pl.pallas_call(
        _sample_body,
        grid=(_N // _RB,),
        in_specs=[
            pl.BlockSpec((_RB, _C), lambda i: (i, 0)),
            pl.BlockSpec((_RB, _C), lambda i: (i, 0)),
        ],
        out_specs=pl.BlockSpec((_RB, _C), lambda i: (i, 0)),
        out_shape=jax.ShapeDtypeStruct((_N, _C), jnp.float32),
    )(probs, gnoise)
    return probs, onehot


# speedup vs baseline: 1.3503x; 1.0986x over previous
import jax
import jax.numpy as jnp
from jax.experimental import pallas as pl
from jax.experimental.pallas import tpu as pltpu

_N = 16384
_C = 100
_T1 = 1001
_TB = 91
_RB = 2048
_TS = 128
_S = 4


def _seg_body(off_ref, x0s_ref, q_ref, probs_ref):
    step = pl.program_id(0)

    def seggroup(jj, carry):
        j0 = _S * jj
        t0 = step * _TB + j0
        nvalid = jnp.minimum(_S, _TB - j0)
        o = [off_ref[t0]]
        for k in range(_S):
            idx = t0 + jnp.minimum(k + 1, nvalid)
            o.append(off_ref[idx])
        qs = [q_ref[jnp.minimum(j0 + k, _TB - 1)] for k in range(_S)]
        start = (o[0] // _TS) * _TS
        end = o[_S]
        ntiles = (end - start + _TS - 1) // _TS

        def tile(k, c2):
            base = start + k * _TS
            xt = x0s_ref[pl.ds(base, _TS), :]
            rows = base + jax.lax.broadcasted_iota(jnp.int32, (_TS, 1), 0)
            acc = None
            for s in range(_S):
                p = jnp.dot(xt, qs[s], preferred_element_type=jnp.float32)
                m = (rows >= o[s]) & (rows < o[s + 1])
                contrib = jnp.where(m, p, 0.0)
                acc = contrib if acc is None else acc + contrib
            union = (rows >= o[0]) & (rows < o[_S])
            old = probs_ref[pl.ds(base, _TS), :]
            probs_ref[pl.ds(base, _TS), :] = jnp.where(union, acc, old)
            return c2

        jax.lax.fori_loop(0, ntiles, tile, 0)
        return carry

    jax.lax.fori_loop(0, (_TB + _S - 1) // _S, seggroup, 0)


def _sample_body(probs_ref, g_ref, oh_ref):
    p = probs_ref[...]
    pn = p / jnp.sum(p, axis=1, keepdims=True)
    y = jnp.log(jnp.maximum(pn, 1e-30)) + g_ref[...]
    s = jnp.argmax(y, axis=1)
    oh_ref[...] = (jax.lax.broadcasted_iota(jnp.int32, (_RB, _C), 1)
                   == s[:, None]).astype(jnp.float32)


def kernel(x0_batch, time_batch, accumulated_q_matrices):
    t32 = time_batch.astype(jnp.int32)
    gnoise = jax.random.gumbel(jax.random.key(1), (_N, _C), jnp.float32)

    perm = jnp.argsort(t32)
    x0s = jnp.take(x0_batch, perm, axis=0)
    hist = jnp.zeros((_T1,), jnp.int32).at[t32].add(1)
    off = jnp.concatenate([jnp.zeros((1,), jnp.int32),
                           jnp.cumsum(hist, dtype=jnp.int32)])
    inv = jnp.zeros((_N,), jnp.int32).at[perm].set(
        jnp.arange(_N, dtype=jnp.int32))

    probs_s = pl.pallas_call(
        _seg_body,
        grid=(_T1 // _TB,),
        in_specs=[
            pl.BlockSpec((_T1 + 1,), lambda s: (0,), memory_space=pltpu.SMEM),
            pl.BlockSpec((_N, _C), lambda s: (0, 0)),
            pl.BlockSpec((_TB, _C, _C), lambda s: (s, 0, 0)),
        ],
        out_specs=pl.BlockSpec((_N, _C), lambda s: (0, 0)),
        out_shape=jax.ShapeDtypeStruct((_N, _C), jnp.float32),
    )(off, x0s, accumulated_q_matrices)

    probs = jnp.take(probs_s, inv, axis=0)

    onehot = pl.pallas_call(
        _sample_body,
        grid=(_N // _RB,),
        in_specs=[
            pl.BlockSpec((_RB, _C), lambda i: (i, 0)),
            pl.BlockSpec((_RB, _C), lambda i: (i, 0)),
        ],
        out_specs=pl.BlockSpec((_RB, _C), lambda i: (i, 0)),
        out_shape=jax.ShapeDtypeStruct((_N, _C), jnp.float32),
    )(probs, gnoise)
    return probs, onehot
